# Initial kernel scaffold; baseline (speedup 1.0000x reference)
import jax
import jax.numpy as jnp
from jax.experimental import pallas as pl


def kernel(x, edge_index, W, b):
    raise NotImplementedError("write your pallas kernel here")


# speedup vs baseline: 15.3424x; 15.3424x over previous
import functools

import jax
import jax.numpy as jnp
from jax import lax
from jax.experimental import pallas as pl
from jax.experimental.pallas import tpu as pltpu
from jax.experimental.pallas import tpu_sc as plsc

N = 10000
E = 160000
D = 256
H = 128
NT = 16
NC = 2
EPT = E // NT
CK = 80
CH = EPT // CK

_mesh = plsc.VectorSubcoreMesh(core_axis_name="c", subcore_axis_name="s")


@functools.partial(
    pl.kernel,
    out_type=jax.ShapeDtypeStruct((NT, N), jnp.float32),
    mesh=_mesh,
    scratch_types=[
        pltpu.VMEM((EPT,), jnp.int32),
        pltpu.VMEM((N,), jnp.float32),
    ],
    compiler_params=pltpu.CompilerParams(needs_layout_passes=False),
)
def _deg_kernel(dst_hbm, out_hbm, dstbuf, hist):
    c = lax.axis_index("c")
    s = lax.axis_index("s")

    @pl.when(c == 0)
    def _():
        pltpu.sync_copy(dst_hbm.at[s], dstbuf)

        def zero(i, carry):
            hist[pl.ds(i * 16, 16)] = jnp.zeros((16,), jnp.float32)
            return carry

        lax.fori_loop(0, N // 16, zero, 0)

        ones = jnp.ones((16,), jnp.float32)

        def body(r, carry):
            for k in range(5):
                idx = dstbuf[pl.ds(r * CK + k * 16, 16)]
                plsc.addupdate_scatter(hist, [idx], ones)
            return carry

        lax.fori_loop(0, CH, body, 0)
        pltpu.sync_copy(hist, out_hbm.at[s])


def _mm_body(x_ref, w_ref, degp_ref, out_ref):
    deg = 1.0 + jnp.sum(degp_ref[...], axis=1, keepdims=True)
    dis = lax.rsqrt(deg)
    h = jnp.dot(x_ref[...], w_ref[...], preferred_element_type=jnp.float32)
    h = h * dis
    out_ref[0] = h[:, :H]
    out_ref[1] = h[:, H:]


def _mm_call(x, W, degp_t):
    return pl.pallas_call(
        _mm_body,
        grid=(10,),
        in_specs=[
            pl.BlockSpec((N // 10, D), lambda i: (i, 0)),
            pl.BlockSpec((D, D), lambda i: (0, 0)),
            pl.BlockSpec((N // 10, NT), lambda i: (i, 0)),
        ],
        out_specs=pl.BlockSpec((NC, N // 10, H), lambda i: (0, i, 0)),
        out_shape=jax.ShapeDtypeStruct((NC, N, H), jnp.float32),
    )(x, W, degp_t)


@functools.partial(
    pl.kernel,
    out_type=jax.ShapeDtypeStruct((NC, N, H), jnp.float32),
    mesh=_mesh,
    scratch_types=[
        pltpu.VMEM((CH, CK), jnp.int32),
        pltpu.VMEM((CH, CK), jnp.int32),
        pltpu.VMEM((CK, H), jnp.float32),
        pltpu.VMEM_SHARED((N, H), jnp.float32),
        pltpu.SemaphoreType.DMA,
    ],
    compiler_params=pltpu.CompilerParams(needs_layout_passes=False),
)
def _agg_kernel(src2_hbm, dst_hbm, h2_hbm, zero_hbm, out_hbm,
                srcbuf, dstbuf, rows, acc, sem):
    c = lax.axis_index("c")
    s = lax.axis_index("s")

    @pl.when(s == 0)
    def _():
        pltpu.sync_copy(zero_hbm, acc)

    pltpu.sync_copy(src2_hbm.at[c, s], srcbuf)
    pltpu.sync_copy(dst_hbm.at[s], dstbuf)
    plsc.subcore_barrier()

    def body(j, carry):
        pltpu.async_copy(h2_hbm.at[srcbuf.at[j]], rows, sem).wait()
        pltpu.sync_copy(rows, acc.at[dstbuf.at[j]], add=True)
        return carry

    lax.fori_loop(0, CH, body, 0)
    plsc.subcore_barrier()

    @pl.when(s == 0)
    def _():
        pltpu.sync_copy(acc, out_hbm.at[c])


def _ep_body(acc_ref, h2_ref, degp_ref, b_ref, out_ref):
    deg = 1.0 + jnp.sum(degp_ref[...], axis=1, keepdims=True)
    dis = lax.rsqrt(deg)
    af = jnp.concatenate([acc_ref[0], acc_ref[1]], axis=1)
    hf = jnp.concatenate([h2_ref[0], h2_ref[1]], axis=1)
    v = dis * (af + hf) + b_ref[0:1, :]
    out_ref[...] = jnp.maximum(v, 0.0)


def _ep_call(acc, h2, degp_t, b8):
    return pl.pallas_call(
        _ep_body,
        grid=(10,),
        in_specs=[
            pl.BlockSpec((NC, N // 10, H), lambda i: (0, i, 0)),
            pl.BlockSpec((NC, N // 10, H), lambda i: (0, i, 0)),
            pl.BlockSpec((N // 10, NT), lambda i: (i, 0)),
            pl.BlockSpec((8, D), lambda i: (0, 0)),
        ],
        out_specs=pl.BlockSpec((N // 10, D), lambda i: (i, 0)),
        out_shape=jax.ShapeDtypeStruct((N, D), jnp.float32),
    )(acc, h2, degp_t, b8)


@jax.jit
def kernel(x, edge_index, W, b):
    src = edge_index[0].astype(jnp.int32)
    dst = edge_index[1].astype(jnp.int32)

    dst_flat = dst.reshape(NT, EPT)
    dst_r = dst.reshape(NT, CH, CK)
    src2 = (src[None, :] + jnp.array([[0], [N]], jnp.int32)).reshape(
        NC, NT, CH, CK)

    degp = _deg_kernel(dst_flat)
    degp_t = degp.T

    h2 = _mm_call(x, W, degp_t)
    h2_flat = h2.reshape(NC * N, H)

    zeros = jnp.zeros((N, H), jnp.float32)
    acc = _agg_kernel(src2, dst_r, h2_flat, zeros)

    b8 = jnp.broadcast_to(b, (8, D))
    return _ep_call(acc, h2, degp_t, b8)


# speedup vs baseline: 19.9628x; 1.3011x over previous
import functools

import jax
import jax.numpy as jnp
from jax import lax
from jax.experimental import pallas as pl
from jax.experimental.pallas import tpu as pltpu
from jax.experimental.pallas import tpu_sc as plsc

N = 10000
E = 160000
D = 256
H = 128
NT = 16
NC = 2
EPT = E // NT
CK = 100
CH = EPT // CK
NSEG = 2
SCH = CH // NSEG

_mesh = plsc.VectorSubcoreMesh(core_axis_name="c", subcore_axis_name="s")


@functools.partial(
    pl.kernel,
    out_type=jax.ShapeDtypeStruct((NT, N), jnp.float32),
    mesh=_mesh,
    scratch_types=[
        pltpu.VMEM((EPT,), jnp.int32),
        pltpu.VMEM((N,), jnp.float32),
    ],
    compiler_params=pltpu.CompilerParams(needs_layout_passes=False),
)
def _deg_kernel(dst_hbm, out_hbm, dstbuf, hist):
    c = lax.axis_index("c")
    s = lax.axis_index("s")

    @pl.when(c == 0)
    def _():
        pltpu.sync_copy(dst_hbm.at[s], dstbuf)

        def zero(i, carry):
            hist[pl.ds(i * 16, 16)] = jnp.zeros((16,), jnp.float32)
            return carry

        lax.fori_loop(0, N // 16, zero, 0)

        ones = jnp.ones((16,), jnp.float32)

        def body(r, carry):
            for k in range(5):
                idx = dstbuf[pl.ds(r * 80 + k * 16, 16)]
                plsc.addupdate_scatter(hist, [idx], ones)
            return carry

        lax.fori_loop(0, EPT // 80, body, 0)
        pltpu.sync_copy(hist, out_hbm.at[s])


def _mm_body(x_ref, w_ref, degp_ref, out_ref):
    deg = 1.0 + jnp.sum(degp_ref[...], axis=1, keepdims=True)
    dis = lax.rsqrt(deg)
    h = jnp.dot(x_ref[...], w_ref[...], preferred_element_type=jnp.float32)
    h = h * dis
    out_ref[0] = h[:, :H]
    out_ref[1] = h[:, H:]


def _mm_call(x, W, degp_t):
    return pl.pallas_call(
        _mm_body,
        grid=(10,),
        in_specs=[
            pl.BlockSpec((N // 10, D), lambda i: (i, 0)),
            pl.BlockSpec((D, D), lambda i: (0, 0)),
            pl.BlockSpec((N // 10, NT), lambda i: (i, 0)),
        ],
        out_specs=pl.BlockSpec((NC, N // 10, H), lambda i: (0, i, 0)),
        out_shape=jax.ShapeDtypeStruct((NC, N, H), jnp.float32),
    )(x, W, degp_t)


@functools.partial(
    pl.kernel,
    out_type=jax.ShapeDtypeStruct((NC, N, H), jnp.float32),
    mesh=_mesh,
    scratch_types=[
        pltpu.VMEM((SCH, CK), jnp.int32),
        pltpu.VMEM((SCH, CK), jnp.int32),
        pltpu.VMEM((CK, H), jnp.float32),
        pltpu.VMEM((CK, H), jnp.float32),
        pltpu.VMEM_SHARED((N, H), jnp.float32),
        pltpu.SemaphoreType.DMA,
        pltpu.SemaphoreType.DMA,
    ],
    compiler_params=pltpu.CompilerParams(needs_layout_passes=False),
)
def _agg_kernel(src2_hbm, dst_hbm, h2_hbm, zero_hbm, out_hbm,
                srcbuf, dstbuf, rows_a, rows_b, acc, sem_a, sem_b):
    c = lax.axis_index("c")
    s = lax.axis_index("s")

    @pl.when(s == 0)
    def _():
        pltpu.sync_copy(zero_hbm, acc)

    plsc.subcore_barrier()

    def gather(j, rows, sem):
        pltpu.async_copy(h2_hbm.at[srcbuf.at[j]], rows, sem)

    def gwait(j, rows, sem):
        pltpu.make_async_copy(h2_hbm.at[srcbuf.at[j]], rows, sem).wait()

    def scatter(j, rows):
        pltpu.sync_copy(rows, acc.at[dstbuf.at[j]], add=True)

    for seg in range(NSEG):
        pltpu.sync_copy(src2_hbm.at[c, s, seg], srcbuf)
        pltpu.sync_copy(dst_hbm.at[s, seg], dstbuf)
        gather(0, rows_a, sem_a)

        def body(i, carry):
            j0 = 2 * i
            gwait(j0, rows_a, sem_a)
            gather(j0 + 1, rows_b, sem_b)
            scatter(j0, rows_a)
            gwait(j0 + 1, rows_b, sem_b)

            @pl.when(i < SCH // 2 - 1)
            def _():
                gather(j0 + 2, rows_a, sem_a)

            scatter(j0 + 1, rows_b)
            return carry

        lax.fori_loop(0, SCH // 2, body, 0)
    plsc.subcore_barrier()

    @pl.when(s == 0)
    def _():
        pltpu.sync_copy(acc, out_hbm.at[c])


def _ep_body(acc_ref, h2_ref, degp_ref, b_ref, out_ref):
    deg = 1.0 + jnp.sum(degp_ref[...], axis=1, keepdims=True)
    dis = lax.rsqrt(deg)
    af = jnp.concatenate([acc_ref[0], acc_ref[1]], axis=1)
    hf = jnp.concatenate([h2_ref[0], h2_ref[1]], axis=1)
    v = dis * (af + hf) + b_ref[0:1, :]
    out_ref[...] = jnp.maximum(v, 0.0)


def _ep_call(acc, h2, degp_t, b8):
    return pl.pallas_call(
        _ep_body,
        grid=(10,),
        in_specs=[
            pl.BlockSpec((NC, N // 10, H), lambda i: (0, i, 0)),
            pl.BlockSpec((NC, N // 10, H), lambda i: (0, i, 0)),
            pl.BlockSpec((N // 10, NT), lambda i: (i, 0)),
            pl.BlockSpec((8, D), lambda i: (0, 0)),
        ],
        out_specs=pl.BlockSpec((N // 10, D), lambda i: (i, 0)),
        out_shape=jax.ShapeDtypeStruct((N, D), jnp.float32),
    )(acc, h2, degp_t, b8)


@jax.jit
def kernel(x, edge_index, W, b):
    src = edge_index[0].astype(jnp.int32)
    dst = edge_index[1].astype(jnp.int32)

    dst_flat = dst.reshape(NT, EPT)
    dst_r = dst.reshape(NT, NSEG, SCH, CK)
    src2 = (src[None, :] + jnp.array([[0], [N]], jnp.int32)).reshape(
        NC, NT, NSEG, SCH, CK)

    degp = _deg_kernel(dst_flat)
    degp_t = degp.T

    h2 = _mm_call(x, W, degp_t)
    h2_flat = h2.reshape(NC * N, H)

    zeros = jnp.zeros((N, H), jnp.float32)
    acc = _agg_kernel(src2, dst_r, h2_flat, zeros)

    b8 = jnp.broadcast_to(b, (8, D))
    return _ep_call(acc, h2, degp_t, b8)


# speedup vs baseline: 20.0027x; 1.0020x over previous
import functools

import jax
import jax.numpy as jnp
from jax import lax
from jax.experimental import pallas as pl
from jax.experimental.pallas import tpu as pltpu
from jax.experimental.pallas import tpu_sc as plsc

N = 10000
E = 160000
D = 256
H = 128
NT = 16
NC = 2
EPT = E // NT
CK = 100
CH = EPT // CK
NSEG = 2
SCH = CH // NSEG

_mesh = plsc.VectorSubcoreMesh(core_axis_name="c", subcore_axis_name="s")


@functools.partial(
    pl.kernel,
    out_type=jax.ShapeDtypeStruct((NT, N), jnp.float32),
    mesh=_mesh,
    scratch_types=[
        pltpu.VMEM((EPT,), jnp.int32),
        pltpu.VMEM((N,), jnp.float32),
    ],
    compiler_params=pltpu.CompilerParams(needs_layout_passes=False),
)
def _deg_kernel(dst_hbm, out_hbm, dstbuf, hist):
    c = lax.axis_index("c")
    s = lax.axis_index("s")

    @pl.when(c == 0)
    def _():
        pltpu.sync_copy(dst_hbm.at[s], dstbuf)

        def zero(i, carry):
            hist[pl.ds(i * 16, 16)] = jnp.zeros((16,), jnp.float32)
            return carry

        lax.fori_loop(0, N // 16, zero, 0)

        ones = jnp.ones((16,), jnp.float32)

        def body(r, carry):
            for k in range(5):
                idx = dstbuf[pl.ds(r * 80 + k * 16, 16)]
                plsc.addupdate_scatter(hist, [idx], ones)
            return carry

        lax.fori_loop(0, EPT // 80, body, 0)
        pltpu.sync_copy(hist, out_hbm.at[s])


def _mm_body(x_ref, w_ref, degp_ref, out_ref):
    deg = 1.0 + jnp.sum(degp_ref[...], axis=1, keepdims=True)
    dis = lax.rsqrt(deg)
    h = jnp.dot(x_ref[...], w_ref[...], preferred_element_type=jnp.float32)
    h = h * dis
    out_ref[0] = h[:, :H]
    out_ref[1] = h[:, H:]


def _mm_call(x, W, degp_t):
    return pl.pallas_call(
        _mm_body,
        grid=(10,),
        in_specs=[
            pl.BlockSpec((N // 10, D), lambda i: (i, 0)),
            pl.BlockSpec((D, D), lambda i: (0, 0)),
            pl.BlockSpec((N // 10, NT), lambda i: (i, 0)),
        ],
        out_specs=pl.BlockSpec((NC, N // 10, H), lambda i: (0, i, 0)),
        out_shape=jax.ShapeDtypeStruct((NC, N, H), jnp.float32),
    )(x, W, degp_t)


@functools.partial(
    pl.kernel,
    out_type=jax.ShapeDtypeStruct((NC, N, H), jnp.float32),
    mesh=_mesh,
    scratch_types=[
        pltpu.VMEM((SCH, CK), jnp.int32),
        pltpu.VMEM((2 * SCH, CK // 2), jnp.int32),
        pltpu.VMEM((CK, H), jnp.float32),
        pltpu.VMEM((CK, H), jnp.float32),
        pltpu.VMEM_SHARED((N, H), jnp.float32),
        pltpu.SemaphoreType.DMA,
        pltpu.SemaphoreType.DMA,
        pltpu.SemaphoreType.DMA,
    ],
    compiler_params=pltpu.CompilerParams(needs_layout_passes=False),
)
def _agg_kernel(src2_hbm, dst_hbm, h2_hbm, zero_hbm, out_hbm,
                srcbuf, dstbuf, rows_a, rows_b, acc, sem_a, sem_b, sem_s):
    c = lax.axis_index("c")
    s = lax.axis_index("s")

    def gather(j, rows, sem):
        pltpu.async_copy(h2_hbm.at[srcbuf.at[j]], rows, sem)

    def gwait(j, rows, sem):
        pltpu.make_async_copy(h2_hbm.at[srcbuf.at[j]], rows, sem).wait()

    def scatter(j, rows):
        d1 = pltpu.async_copy(rows.at[pl.ds(0, CK // 2)],
                              acc.at[dstbuf.at[2 * j]], sem_s, add=True)
        d2 = pltpu.async_copy(rows.at[pl.ds(CK // 2, CK // 2)],
                              acc.at[dstbuf.at[2 * j + 1]], sem_s, add=True)
        d1.wait()
        d2.wait()

    for seg in range(NSEG):
        pltpu.sync_copy(src2_hbm.at[c, s, seg], srcbuf)
        pltpu.sync_copy(dst_hbm.at[s, seg], dstbuf)
        gather(0, rows_a, sem_a)
        if seg == 0:
            @pl.when(s == 0)
            def _():
                pltpu.sync_copy(zero_hbm, acc)

            plsc.subcore_barrier()

        def body(i, carry):
            j0 = 2 * i
            gwait(j0, rows_a, sem_a)
            gather(j0 + 1, rows_b, sem_b)
            scatter(j0, rows_a)
            gwait(j0 + 1, rows_b, sem_b)

            @pl.when(i < SCH // 2 - 1)
            def _():
                gather(j0 + 2, rows_a, sem_a)

            scatter(j0 + 1, rows_b)
            return carry

        lax.fori_loop(0, SCH // 2, body, 0)
    plsc.subcore_barrier()

    @pl.when(s == 0)
    def _():
        pltpu.sync_copy(acc, out_hbm.at[c])


def _ep_body(acc_ref, h2_ref, degp_ref, b_ref, out_ref):
    deg = 1.0 + jnp.sum(degp_ref[...], axis=1, keepdims=True)
    dis = lax.rsqrt(deg)
    af = jnp.concatenate([acc_ref[0], acc_ref[1]], axis=1)
    hf = jnp.concatenate([h2_ref[0], h2_ref[1]], axis=1)
    v = dis * (af + hf) + b_ref[0:1, :]
    out_ref[...] = jnp.maximum(v, 0.0)


def _ep_call(acc, h2, degp_t, b8):
    return pl.pallas_call(
        _ep_body,
        grid=(10,),
        in_specs=[
            pl.BlockSpec((NC, N // 10, H), lambda i: (0, i, 0)),
            pl.BlockSpec((NC, N // 10, H), lambda i: (0, i, 0)),
            pl.BlockSpec((N // 10, NT), lambda i: (i, 0)),
            pl.BlockSpec((8, D), lambda i: (0, 0)),
        ],
        out_specs=pl.BlockSpec((N // 10, D), lambda i: (i, 0)),
        out_shape=jax.ShapeDtypeStruct((N, D), jnp.float32),
    )(acc, h2, degp_t, b8)


@jax.jit
def kernel(x, edge_index, W, b):
    src = edge_index[0].astype(jnp.int32)
    dst = edge_index[1].astype(jnp.int32)

    dst_flat = dst.reshape(NT, EPT)
    dst_r = dst.reshape(NT, NSEG, 2 * SCH, CK // 2)
    src2 = (src[None, :] + jnp.array([[0], [N]], jnp.int32)).reshape(
        NC, NT, NSEG, SCH, CK)

    degp = _deg_kernel(dst_flat)
    degp_t = degp.T

    h2 = _mm_call(x, W, degp_t)
    h2_flat = h2.reshape(NC * N, H)

    zeros = jnp.zeros((N, H), jnp.float32)
    acc = _agg_kernel(src2, dst_r, h2_flat, zeros)

    b8 = jnp.broadcast_to(b, (8, D))
    return _ep_call(acc, h2, degp_t, b8)


# speedup vs baseline: 21.2708x; 1.0634x over previous
import functools

import jax
import jax.numpy as jnp
from jax import lax
from jax.experimental import pallas as pl
from jax.experimental.pallas import tpu as pltpu
from jax.experimental.pallas import tpu_sc as plsc

N = 10000
E = 160000
D = 256
H = 128
NT = 16
NC = 2
EPT = E // NT
CK = 125
CH = EPT // CK
NSEG = 2
SCH = CH // NSEG

_mesh = plsc.VectorSubcoreMesh(core_axis_name="c", subcore_axis_name="s")


@functools.partial(
    pl.kernel,
    out_type=jax.ShapeDtypeStruct((NT, N), jnp.float32),
    mesh=_mesh,
    scratch_types=[
        pltpu.VMEM((EPT,), jnp.int32),
        pltpu.VMEM((N,), jnp.float32),
    ],
    compiler_params=pltpu.CompilerParams(needs_layout_passes=False),
)
def _deg_kernel(dst_hbm, out_hbm, dstbuf, hist):
    c = lax.axis_index("c")
    s = lax.axis_index("s")

    @pl.when(c == 0)
    def _():
        pltpu.sync_copy(dst_hbm.at[s], dstbuf)

        def zero(i, carry):
            hist[pl.ds(i * 16, 16)] = jnp.zeros((16,), jnp.float32)
            return carry

        lax.fori_loop(0, N // 16, zero, 0)

        ones = jnp.ones((16,), jnp.float32)

        def body(r, carry):
            for k in range(5):
                idx = dstbuf[pl.ds(r * 80 + k * 16, 16)]
                plsc.addupdate_scatter(hist, [idx], ones)
            return carry

        lax.fori_loop(0, EPT // 80, body, 0)
        pltpu.sync_copy(hist, out_hbm.at[s])


def _mm_body(x_ref, w_ref, degp_ref, out_ref):
    deg = 1.0 + jnp.sum(degp_ref[...], axis=1, keepdims=True)
    dis = lax.rsqrt(deg)
    h = jnp.dot(x_ref[...], w_ref[...], preferred_element_type=jnp.float32)
    h = h * dis
    out_ref[0] = h[:, :H]
    out_ref[1] = h[:, H:]


def _mm_call(x, W, degp_t):
    return pl.pallas_call(
        _mm_body,
        grid=(10,),
        in_specs=[
            pl.BlockSpec((N // 10, D), lambda i: (i, 0)),
            pl.BlockSpec((D, D), lambda i: (0, 0)),
            pl.BlockSpec((N // 10, NT), lambda i: (i, 0)),
        ],
        out_specs=pl.BlockSpec((NC, N // 10, H), lambda i: (0, i, 0)),
        out_shape=jax.ShapeDtypeStruct((NC, N, H), jnp.float32),
    )(x, W, degp_t)


@functools.partial(
    pl.kernel,
    out_type=jax.ShapeDtypeStruct((NC, N, H), jnp.float32),
    mesh=_mesh,
    scratch_types=[
        pltpu.VMEM((SCH, CK), jnp.int32),
        pltpu.VMEM((SCH, CK), jnp.int32),
        pltpu.VMEM((CK, H), jnp.float32),
        pltpu.VMEM((CK, H), jnp.float32),
        pltpu.VMEM_SHARED((N, H), jnp.float32),
        pltpu.SemaphoreType.DMA,
        pltpu.SemaphoreType.DMA,
    ],
    compiler_params=pltpu.CompilerParams(needs_layout_passes=False),
)
def _agg_kernel(src2_hbm, dst_hbm, h2_hbm, zero_hbm, out_hbm,
                srcbuf, dstbuf, rows_a, rows_b, acc, sem_a, sem_b):
    c = lax.axis_index("c")
    s = lax.axis_index("s")

    def gather(j, rows, sem):
        pltpu.async_copy(h2_hbm.at[srcbuf.at[j]], rows, sem)

    def gwait(j, rows, sem):
        pltpu.make_async_copy(h2_hbm.at[srcbuf.at[j]], rows, sem).wait()

    def scatter(j, rows):
        pltpu.sync_copy(rows, acc.at[dstbuf.at[j]], add=True)

    for seg in range(NSEG):
        pltpu.sync_copy(src2_hbm.at[c, s, seg], srcbuf)
        pltpu.sync_copy(dst_hbm.at[s, seg], dstbuf)
        gather(0, rows_a, sem_a)
        if seg == 0:
            @pl.when(s == 0)
            def _():
                pltpu.sync_copy(zero_hbm, acc)

            plsc.subcore_barrier()

        def body(i, carry):
            j0 = 2 * i
            gwait(j0, rows_a, sem_a)
            gather(j0 + 1, rows_b, sem_b)
            scatter(j0, rows_a)
            gwait(j0 + 1, rows_b, sem_b)

            @pl.when(i < SCH // 2 - 1)
            def _():
                gather(j0 + 2, rows_a, sem_a)

            scatter(j0 + 1, rows_b)
            return carry

        lax.fori_loop(0, SCH // 2, body, 0)
    plsc.subcore_barrier()

    @pl.when(s == 0)
    def _():
        pltpu.sync_copy(acc, out_hbm.at[c])


def _ep_body(acc_ref, h2_ref, degp_ref, b_ref, out_ref):
    deg = 1.0 + jnp.sum(degp_ref[...], axis=1, keepdims=True)
    dis = lax.rsqrt(deg)
    af = jnp.concatenate([acc_ref[0], acc_ref[1]], axis=1)
    hf = jnp.concatenate([h2_ref[0], h2_ref[1]], axis=1)
    v = dis * (af + hf) + b_ref[0:1, :]
    out_ref[...] = jnp.maximum(v, 0.0)


def _ep_call(acc, h2, degp_t, b8):
    return pl.pallas_call(
        _ep_body,
        grid=(10,),
        in_specs=[
            pl.BlockSpec((NC, N // 10, H), lambda i: (0, i, 0)),
            pl.BlockSpec((NC, N // 10, H), lambda i: (0, i, 0)),
            pl.BlockSpec((N // 10, NT), lambda i: (i, 0)),
            pl.BlockSpec((8, D), lambda i: (0, 0)),
        ],
        out_specs=pl.BlockSpec((N // 10, D), lambda i: (i, 0)),
        out_shape=jax.ShapeDtypeStruct((N, D), jnp.float32),
    )(acc, h2, degp_t, b8)


@jax.jit
def kernel(x, edge_index, W, b):
    src = edge_index[0].astype(jnp.int32)
    dst = edge_index[1].astype(jnp.int32)

    dst_flat = dst.reshape(NT, EPT)
    dst_r = dst.reshape(NT, NSEG, SCH, CK)
    src2 = (src[None, :] + jnp.array([[0], [N]], jnp.int32)).reshape(
        NC, NT, NSEG, SCH, CK)

    degp = _deg_kernel(dst_flat)
    degp_t = degp.T

    h2 = _mm_call(x, W, degp_t)
    h2_flat = h2.reshape(NC * N, H)

    zeros = jnp.zeros((N, H), jnp.float32)
    acc = _agg_kernel(src2, dst_r, h2_flat, zeros)

    b8 = jnp.broadcast_to(b, (8, D))
    return _ep_call(acc, h2, degp_t, b8)
